# Initial kernel scaffold; baseline (speedup 1.0000x reference)
import jax
import jax.numpy as jnp
from jax.experimental import pallas as pl


def kernel(x, edge_index, edge_type, W1, Wroot1, W2, Wroot2):
    raise NotImplementedError("write your pallas kernel here")


# speedup vs baseline: 80.4434x; 80.4434x over previous
import functools

import jax
import jax.numpy as jnp
from jax import lax
from jax.experimental import pallas as pl
from jax.experimental.pallas import tpu as pltpu
from jax.experimental.pallas import tpu_sc as plsc

N = 10000
E = 320000
R = 16
D = 128
H = 128
O = 16
NR = N * R
NC = 2
NS = 16
NW = NC * NS
NP = 10240

_CH = 80
_ZC = 2000


def _i32(v):
    return jnp.int32(v)


def _make_mesh():
    return plsc.VectorSubcoreMesh(core_axis_name="c", subcore_axis_name="s")


def _hist_norm(seg):
    EH = E // NS
    EW = E // NW
    ZR = NR // NS

    @functools.partial(
        pl.kernel,
        out_type=jax.ShapeDtypeStruct((E,), jnp.float32),
        mesh=_make_mesh(),
        scratch_types=[
            pltpu.VMEM_SHARED((NR,), jnp.float32),
            pltpu.VMEM((_CH,), jnp.int32),
            pltpu.VMEM((_CH,), jnp.float32),
            pltpu.VMEM((_CH,), jnp.float32),
            pltpu.VMEM((_ZC,), jnp.float32),
        ],
    )
    def k(seg_hbm, norm_hbm, cnt_sh, segbuf, ones, nbuf, zbuf):
        c = lax.axis_index("c")
        s = lax.axis_index("s")
        w = s * _i32(NC) + c

        for i in range(_CH // 16):
            ones[pl.ds(i * 16, 16)] = jnp.ones((16,), jnp.float32)

        @pl.loop(_i32(0), _i32(_ZC // 16))
        def _(i):
            zbuf[pl.ds(i * _i32(16), 16)] = jnp.zeros((16,), jnp.float32)

        @pl.loop(_i32(0), _i32(ZR // _ZC))
        def _(i):
            pltpu.sync_copy(zbuf, cnt_sh.at[pl.ds(s * _i32(ZR) + i * _i32(_ZC), _ZC)])

        plsc.subcore_barrier()

        @pl.loop(_i32(0), _i32(EH // _CH))
        def _(i):
            base = s * _i32(EH) + i * _i32(_CH)
            pltpu.sync_copy(seg_hbm.at[pl.ds(base, _CH)], segbuf)
            pltpu.sync_copy(ones, cnt_sh.at[segbuf], add=True)

        plsc.subcore_barrier()

        @pl.loop(_i32(0), _i32(EW // _CH))
        def _(i):
            base = w * _i32(EW) + i * _i32(_CH)
            pltpu.sync_copy(seg_hbm.at[pl.ds(base, _CH)], segbuf)
            pltpu.sync_copy(cnt_sh.at[segbuf], nbuf)

            for j in range(_CH // 16):
                v = nbuf[pl.ds(j * 16, 16)]
                nbuf[pl.ds(j * 16, 16)] = 1.0 / jnp.maximum(v, 1.0)

            pltpu.sync_copy(nbuf, norm_hbm.at[pl.ds(base, _CH)])

    return k(seg)


def _edge_agg(ridx, dst, norm, table, width):
    EW = E // NW
    RPT = NP // NS
    ZB = 80
    nv = width // 16

    @functools.partial(
        pl.kernel,
        out_type=jax.ShapeDtypeStruct((NC, NP, width), jnp.float32),
        mesh=_make_mesh(),
        scratch_types=[
            pltpu.VMEM_SHARED((NP, width), jnp.float32),
            pltpu.VMEM((_CH,), jnp.int32),
            pltpu.VMEM((_CH,), jnp.int32),
            pltpu.VMEM((_CH + 16,), jnp.float32),
            pltpu.VMEM((_CH, width), jnp.float32),
            pltpu.VMEM((ZB, width), jnp.float32),
        ],
    )
    def k(ridx_hbm, dst_hbm, norm_hbm, tab_hbm, out_hbm,
          agg_sh, ribuf, dibuf, nbuf, rows, zbuf):
        c = lax.axis_index("c")
        s = lax.axis_index("s")
        w = s * _i32(NC) + c

        @pl.loop(_i32(0), _i32(ZB))
        def _(i):
            for j in range(nv):
                zbuf[i, pl.ds(j * 16, 16)] = jnp.zeros((16,), jnp.float32)

        @pl.loop(_i32(0), _i32(RPT // ZB))
        def _(i):
            pltpu.sync_copy(zbuf, agg_sh.at[pl.ds(s * _i32(RPT) + i * _i32(ZB), ZB)])

        plsc.subcore_barrier()

        @pl.loop(_i32(0), _i32(EW // _CH))
        def _(i):
            base = w * _i32(EW) + i * _i32(_CH)
            pltpu.sync_copy(ridx_hbm.at[pl.ds(base, _CH)], ribuf)
            pltpu.sync_copy(dst_hbm.at[pl.ds(base, _CH)], dibuf)
            pltpu.sync_copy(norm_hbm.at[pl.ds(base, _CH)], nbuf.at[pl.ds(0, _CH)])
            pltpu.sync_copy(tab_hbm.at[ribuf], rows)

            @pl.loop(_i32(0), _i32(_CH))
            def _(e):
                sc = nbuf[pl.ds(e, 16)][0]
                for j in range(nv):
                    rows[e, pl.ds(j * 16, 16)] = rows[e, pl.ds(j * 16, 16)] * sc

            pltpu.sync_copy(rows, agg_sh.at[dibuf], add=True)

        plsc.subcore_barrier()

        @pl.loop(_i32(0), _i32(RPT // ZB))
        def _(i):
            r0 = s * _i32(RPT) + i * _i32(ZB)
            pltpu.sync_copy(agg_sh.at[pl.ds(r0, ZB)], out_hbm.at[c, pl.ds(r0, ZB)])

    return k(ridx, dst, norm, table)


def _edge_agg16(gidx, sub, dst, norm, table):
    EW = E // NW
    RPT = NP // NS
    ZB = 80
    WV = 128

    @functools.partial(
        pl.kernel,
        out_type=jax.ShapeDtypeStruct((NC, NP, WV), jnp.float32),
        mesh=_make_mesh(),
        scratch_types=[
            pltpu.VMEM_SHARED((NP, WV), jnp.float32),
            pltpu.VMEM((_CH,), jnp.int32),
            pltpu.VMEM((_CH,), jnp.int32),
            pltpu.VMEM((_CH + 16,), jnp.int32),
            pltpu.VMEM((_CH + 16,), jnp.float32),
            pltpu.VMEM((_CH, 128), jnp.float32),
            pltpu.VMEM((_CH, WV), jnp.float32),
            pltpu.VMEM((ZB, WV), jnp.float32),
        ],
    )
    def k(gidx_hbm, sub_hbm, dst_hbm, norm_hbm, tab_hbm, out_hbm,
          agg_sh, gibuf, dibuf, subbuf, nbuf, rows, cbuf, zbuf):
        c = lax.axis_index("c")
        s = lax.axis_index("s")
        w = s * _i32(NC) + c

        @pl.loop(_i32(0), _i32(ZB))
        def _(i):
            for j in range(WV // 16):
                zbuf[i, pl.ds(j * 16, 16)] = jnp.zeros((16,), jnp.float32)

        @pl.loop(_i32(0), _i32(_CH))
        def _(i):
            for j in range(WV // 16):
                cbuf[i, pl.ds(j * 16, 16)] = jnp.zeros((16,), jnp.float32)

        @pl.loop(_i32(0), _i32(RPT // ZB))
        def _(i):
            pltpu.sync_copy(zbuf, agg_sh.at[pl.ds(s * _i32(RPT) + i * _i32(ZB), ZB)])

        plsc.subcore_barrier()

        @pl.loop(_i32(0), _i32(EW // _CH))
        def _(i):
            base = w * _i32(EW) + i * _i32(_CH)
            pltpu.sync_copy(gidx_hbm.at[pl.ds(base, _CH)], gibuf)
            pltpu.sync_copy(dst_hbm.at[pl.ds(base, _CH)], dibuf)
            pltpu.sync_copy(sub_hbm.at[pl.ds(base, _CH)], subbuf.at[pl.ds(0, _CH)])
            pltpu.sync_copy(norm_hbm.at[pl.ds(base, _CH)], nbuf.at[pl.ds(0, _CH)])
            pltpu.sync_copy(tab_hbm.at[gibuf], rows)

            @pl.loop(_i32(0), _i32(_CH))
            def _(e):
                sc = nbuf[pl.ds(e, 16)][0]
                off = subbuf[pl.ds(e, 16)][0]
                cbuf[e, pl.ds(0, 16)] = rows[e, pl.ds(off, 16)] * sc

            pltpu.sync_copy(cbuf, agg_sh.at[dibuf], add=True)

        plsc.subcore_barrier()

        @pl.loop(_i32(0), _i32(RPT // ZB))
        def _(i):
            r0 = s * _i32(RPT) + i * _i32(ZB)
            pltpu.sync_copy(agg_sh.at[pl.ds(r0, ZB)], out_hbm.at[c, pl.ds(r0, ZB)])

    return k(gidx, sub, dst, norm, table)


def _mm2(xin, wa, wb, nb):
    n, d = xin.shape
    ka, kb = wa.shape[1], wb.shape[1]

    def body(x_ref, wa_ref, wb_ref, o1_ref, o2_ref):
        xv = x_ref[...]
        o1_ref[...] = jnp.dot(xv, wa_ref[...], preferred_element_type=jnp.float32)
        o2_ref[...] = jnp.dot(xv, wb_ref[...], preferred_element_type=jnp.float32)

    z = lambda: jnp.int32(0)
    return pl.pallas_call(
        body,
        grid=(n // nb,),
        in_specs=[pl.BlockSpec((nb, d), lambda i: (i, z())),
                  pl.BlockSpec((d, ka), lambda i: (z(), z())),
                  pl.BlockSpec((d, kb), lambda i: (z(), z()))],
        out_specs=[pl.BlockSpec((nb, ka), lambda i: (i, z())),
                   pl.BlockSpec((nb, kb), lambda i: (i, z()))],
        out_shape=[jax.ShapeDtypeStruct((n, ka), jnp.float32),
                   jax.ShapeDtypeStruct((n, kb), jnp.float32)],
    )(xin, wa, wb)


def _relu_mm2(parts, xroot, wa, wb, nb):
    n, d = xroot.shape
    ka, kb = wa.shape[1], wb.shape[1]

    def body(p_ref, xr_ref, wa_ref, wb_ref, o1_ref, o2_ref):
        h = jnp.maximum(p_ref[0] + p_ref[1] + xr_ref[...], 0.0)
        o1_ref[...] = jnp.dot(h, wa_ref[...], preferred_element_type=jnp.float32)
        o2_ref[...] = jnp.dot(h, wb_ref[...], preferred_element_type=jnp.float32)

    z = lambda: jnp.int32(0)
    return pl.pallas_call(
        body,
        grid=(n // nb,),
        in_specs=[pl.BlockSpec((NC, nb, d), lambda i: (z(), i, z())),
                  pl.BlockSpec((nb, d), lambda i: (i, z())),
                  pl.BlockSpec((d, ka), lambda i: (z(), z())),
                  pl.BlockSpec((d, kb), lambda i: (z(), z()))],
        out_specs=[pl.BlockSpec((nb, ka), lambda i: (i, z())),
                   pl.BlockSpec((nb, kb), lambda i: (i, z()))],
        out_shape=[jax.ShapeDtypeStruct((n, ka), jnp.float32),
                   jax.ShapeDtypeStruct((n, kb), jnp.float32)],
    )(parts, xroot, wa, wb)


def _combine(parts, hroot, nb):
    n, d = hroot.shape

    def body(p_ref, hr_ref, o_ref):
        o_ref[...] = p_ref[0] + p_ref[1] + hr_ref[...]

    z = lambda: jnp.int32(0)
    return pl.pallas_call(
        body,
        grid=(n // nb,),
        in_specs=[pl.BlockSpec((NC, nb, d), lambda i: (z(), i, z())),
                  pl.BlockSpec((nb, d), lambda i: (i, z()))],
        out_specs=pl.BlockSpec((nb, d), lambda i: (i, z())),
        out_shape=jax.ShapeDtypeStruct((n, d), jnp.float32),
    )(parts, hroot)


def _jax_hist_norm(seg):
    cnt = jax.ops.segment_sum(jnp.ones((E,), jnp.float32), seg, num_segments=NR)
    return 1.0 / jnp.maximum(cnt[seg], 1.0)


def _jax_edge_agg(ridx, dst, norm, table, width):
    msg = table[ridx] * norm[:, None].astype(jnp.float32)
    agg = jax.ops.segment_sum(msg, dst, num_segments=NP)
    out = jnp.zeros((NC, NP, width), jnp.float32).at[0].set(agg)
    return out


def kernel(x, edge_index, edge_type, W1, Wroot1, W2, Wroot2):
    x = x.astype(jnp.float32)
    src = edge_index[0].astype(jnp.int32)
    dst = edge_index[1].astype(jnp.int32)
    et = edge_type.astype(jnp.int32)
    ridx = src * R + et
    seg = dst * R + et

    w1c = W1.astype(jnp.float32).transpose(1, 0, 2).reshape(D, R * H)
    w2c = W2.astype(jnp.float32).transpose(1, 0, 2).reshape(H, R * O)

    norm = _hist_norm(seg)
    hr1, xr = _mm2(x, w1c, Wroot1.astype(jnp.float32), 1000)
    agg1p = _edge_agg(ridx, dst, norm, hr1.reshape(NR, H), H)
    hr2, hroot = _relu_mm2(agg1p[:, :N], xr, w2c, Wroot2.astype(jnp.float32), 1000)
    gidx = lax.shift_right_logical(ridx, jnp.int32(3))
    sub = (ridx & jnp.int32(7)) * jnp.int32(O)
    agg2p = _edge_agg16(gidx, sub, dst, norm, hr2.reshape(NR // 8, 8 * O))
    return _combine(agg2p[:, :N, :O], hroot, 1000).astype(jnp.float64)


# speedup vs baseline: 89.8958x; 1.1175x over previous
import functools

import jax
import jax.numpy as jnp
from jax import lax
from jax.experimental import pallas as pl
from jax.experimental.pallas import tpu as pltpu
from jax.experimental.pallas import tpu_sc as plsc

N = 10000
E = 320000
R = 16
D = 128
H = 128
O = 16
NR = N * R
NC = 2
NS = 16
NW = NC * NS
NP = 10240

_CH = 80
_ZC = 2000


def _i32(v):
    return jnp.int32(v)


def _make_mesh():
    return plsc.VectorSubcoreMesh(core_axis_name="c", subcore_axis_name="s")


def _hist_norm(seg):
    EH = E // NS
    EW = E // NW
    ZR = NR // NS

    @functools.partial(
        pl.kernel,
        out_type=jax.ShapeDtypeStruct((E,), jnp.float32),
        mesh=_make_mesh(),
        scratch_types=[
            pltpu.VMEM_SHARED((NR,), jnp.float32),
            pltpu.VMEM((_CH,), jnp.int32),
            pltpu.VMEM((_CH,), jnp.float32),
            pltpu.VMEM((_CH,), jnp.float32),
            pltpu.VMEM((_ZC,), jnp.float32),
        ],
    )
    def k(seg_hbm, norm_hbm, cnt_sh, segbuf, ones, nbuf, zbuf):
        c = lax.axis_index("c")
        s = lax.axis_index("s")
        w = s * _i32(NC) + c

        for i in range(_CH // 16):
            ones[pl.ds(i * 16, 16)] = jnp.ones((16,), jnp.float32)

        @pl.loop(_i32(0), _i32(_ZC // 16))
        def _(i):
            zbuf[pl.ds(i * _i32(16), 16)] = jnp.zeros((16,), jnp.float32)

        @pl.loop(_i32(0), _i32(ZR // _ZC))
        def _(i):
            pltpu.sync_copy(zbuf, cnt_sh.at[pl.ds(s * _i32(ZR) + i * _i32(_ZC), _ZC)])

        plsc.subcore_barrier()

        @pl.loop(_i32(0), _i32(EH // _CH))
        def _(i):
            base = s * _i32(EH) + i * _i32(_CH)
            pltpu.sync_copy(seg_hbm.at[pl.ds(base, _CH)], segbuf)
            pltpu.sync_copy(ones, cnt_sh.at[segbuf], add=True)

        plsc.subcore_barrier()

        @pl.loop(_i32(0), _i32(EW // _CH))
        def _(i):
            base = w * _i32(EW) + i * _i32(_CH)
            pltpu.sync_copy(seg_hbm.at[pl.ds(base, _CH)], segbuf)
            pltpu.sync_copy(cnt_sh.at[segbuf], nbuf)

            for j in range(_CH // 16):
                v = nbuf[pl.ds(j * 16, 16)]
                nbuf[pl.ds(j * 16, 16)] = 1.0 / jnp.maximum(v, 1.0)

            pltpu.sync_copy(nbuf, norm_hbm.at[pl.ds(base, _CH)])

    return k(seg)


def _edge_agg(ridx, dst, norm, table, width):
    EW = E // NW
    RPT = NP // NS
    ZB = 80
    nv = width // 16

    @functools.partial(
        pl.kernel,
        out_type=jax.ShapeDtypeStruct((NC, NP, width), jnp.float32),
        mesh=_make_mesh(),
        scratch_types=[
            pltpu.VMEM_SHARED((NP, width), jnp.float32),
            pltpu.VMEM((_CH,), jnp.int32),
            pltpu.VMEM((_CH,), jnp.int32),
            pltpu.VMEM((_CH + 16,), jnp.float32),
            pltpu.VMEM((_CH, width), jnp.float32),
            pltpu.VMEM((ZB, width), jnp.float32),
        ],
    )
    def k(ridx_hbm, dst_hbm, norm_hbm, tab_hbm, out_hbm,
          agg_sh, ribuf, dibuf, nbuf, rows, zbuf):
        c = lax.axis_index("c")
        s = lax.axis_index("s")
        w = s * _i32(NC) + c

        @pl.loop(_i32(0), _i32(ZB))
        def _(i):
            for j in range(nv):
                zbuf[i, pl.ds(j * 16, 16)] = jnp.zeros((16,), jnp.float32)

        @pl.loop(_i32(0), _i32(RPT // ZB))
        def _(i):
            pltpu.sync_copy(zbuf, agg_sh.at[pl.ds(s * _i32(RPT) + i * _i32(ZB), ZB)])

        plsc.subcore_barrier()

        @pl.loop(_i32(0), _i32(EW // _CH))
        def _(i):
            base = w * _i32(EW) + i * _i32(_CH)
            pltpu.sync_copy(ridx_hbm.at[pl.ds(base, _CH)], ribuf)
            pltpu.sync_copy(dst_hbm.at[pl.ds(base, _CH)], dibuf)
            pltpu.sync_copy(norm_hbm.at[pl.ds(base, _CH)], nbuf.at[pl.ds(0, _CH)])
            pltpu.sync_copy(tab_hbm.at[ribuf], rows)

            for e in range(_CH):
                sc = nbuf[pl.ds(e, 16)][0]
                for j in range(nv):
                    rows[e, pl.ds(j * 16, 16)] = rows[e, pl.ds(j * 16, 16)] * sc

            pltpu.sync_copy(rows, agg_sh.at[dibuf], add=True)

        plsc.subcore_barrier()

        @pl.loop(_i32(0), _i32(RPT // ZB))
        def _(i):
            r0 = s * _i32(RPT) + i * _i32(ZB)
            pltpu.sync_copy(agg_sh.at[pl.ds(r0, ZB)], out_hbm.at[c, pl.ds(r0, ZB)])

    return k(ridx, dst, norm, table)


def _edge_agg16(gidx, sub, dst, norm, table):
    EW = E // NW
    RPT = NP // NS
    ZB = 80
    WV = 128

    @functools.partial(
        pl.kernel,
        out_type=jax.ShapeDtypeStruct((NC, NP, WV), jnp.float32),
        mesh=_make_mesh(),
        scratch_types=[
            pltpu.VMEM_SHARED((NP, WV), jnp.float32),
            pltpu.VMEM((_CH,), jnp.int32),
            pltpu.VMEM((_CH,), jnp.int32),
            pltpu.VMEM((_CH + 16,), jnp.int32),
            pltpu.VMEM((_CH + 16,), jnp.float32),
            pltpu.VMEM((_CH, 128), jnp.float32),
            pltpu.VMEM((_CH, WV), jnp.float32),
            pltpu.VMEM((ZB, WV), jnp.float32),
        ],
    )
    def k(gidx_hbm, sub_hbm, dst_hbm, norm_hbm, tab_hbm, out_hbm,
          agg_sh, gibuf, dibuf, subbuf, nbuf, rows, cbuf, zbuf):
        c = lax.axis_index("c")
        s = lax.axis_index("s")
        w = s * _i32(NC) + c

        @pl.loop(_i32(0), _i32(ZB))
        def _(i):
            for j in range(WV // 16):
                zbuf[i, pl.ds(j * 16, 16)] = jnp.zeros((16,), jnp.float32)

        @pl.loop(_i32(0), _i32(_CH))
        def _(i):
            for j in range(WV // 16):
                cbuf[i, pl.ds(j * 16, 16)] = jnp.zeros((16,), jnp.float32)

        @pl.loop(_i32(0), _i32(RPT // ZB))
        def _(i):
            pltpu.sync_copy(zbuf, agg_sh.at[pl.ds(s * _i32(RPT) + i * _i32(ZB), ZB)])

        plsc.subcore_barrier()

        @pl.loop(_i32(0), _i32(EW // _CH))
        def _(i):
            base = w * _i32(EW) + i * _i32(_CH)
            pltpu.sync_copy(gidx_hbm.at[pl.ds(base, _CH)], gibuf)
            pltpu.sync_copy(dst_hbm.at[pl.ds(base, _CH)], dibuf)
            pltpu.sync_copy(sub_hbm.at[pl.ds(base, _CH)], subbuf.at[pl.ds(0, _CH)])
            pltpu.sync_copy(norm_hbm.at[pl.ds(base, _CH)], nbuf.at[pl.ds(0, _CH)])
            pltpu.sync_copy(tab_hbm.at[gibuf], rows)

            for e in range(_CH):
                sc = nbuf[pl.ds(e, 16)][0]
                off = subbuf[pl.ds(e, 16)][0]
                cbuf[e, pl.ds(0, 16)] = rows[e, pl.ds(off, 16)] * sc

            pltpu.sync_copy(cbuf, agg_sh.at[dibuf], add=True)

        plsc.subcore_barrier()

        @pl.loop(_i32(0), _i32(RPT // ZB))
        def _(i):
            r0 = s * _i32(RPT) + i * _i32(ZB)
            pltpu.sync_copy(agg_sh.at[pl.ds(r0, ZB)], out_hbm.at[c, pl.ds(r0, ZB)])

    return k(gidx, sub, dst, norm, table)


def _mm2(xin, wa, wb, nb):
    n, d = xin.shape
    ka, kb = wa.shape[1], wb.shape[1]

    def body(x_ref, wa_ref, wb_ref, o1_ref, o2_ref):
        xv = x_ref[...]
        o1_ref[...] = jnp.dot(xv, wa_ref[...], preferred_element_type=jnp.float32)
        o2_ref[...] = jnp.dot(xv, wb_ref[...], preferred_element_type=jnp.float32)

    z = lambda: jnp.int32(0)
    return pl.pallas_call(
        body,
        grid=(n // nb,),
        in_specs=[pl.BlockSpec((nb, d), lambda i: (i, z())),
                  pl.BlockSpec((d, ka), lambda i: (z(), z())),
                  pl.BlockSpec((d, kb), lambda i: (z(), z()))],
        out_specs=[pl.BlockSpec((nb, ka), lambda i: (i, z())),
                   pl.BlockSpec((nb, kb), lambda i: (i, z()))],
        out_shape=[jax.ShapeDtypeStruct((n, ka), jnp.float32),
                   jax.ShapeDtypeStruct((n, kb), jnp.float32)],
    )(xin, wa, wb)


def _relu_mm2(parts, xroot, wa, wb, nb):
    n, d = xroot.shape
    ka, kb = wa.shape[1], wb.shape[1]

    def body(p_ref, xr_ref, wa_ref, wb_ref, o1_ref, o2_ref):
        h = jnp.maximum(p_ref[0] + p_ref[1] + xr_ref[...], 0.0)
        o1_ref[...] = jnp.dot(h, wa_ref[...], preferred_element_type=jnp.float32)
        o2_ref[...] = jnp.dot(h, wb_ref[...], preferred_element_type=jnp.float32)

    z = lambda: jnp.int32(0)
    return pl.pallas_call(
        body,
        grid=(n // nb,),
        in_specs=[pl.BlockSpec((NC, nb, d), lambda i: (z(), i, z())),
                  pl.BlockSpec((nb, d), lambda i: (i, z())),
                  pl.BlockSpec((d, ka), lambda i: (z(), z())),
                  pl.BlockSpec((d, kb), lambda i: (z(), z()))],
        out_specs=[pl.BlockSpec((nb, ka), lambda i: (i, z())),
                   pl.BlockSpec((nb, kb), lambda i: (i, z()))],
        out_shape=[jax.ShapeDtypeStruct((n, ka), jnp.float32),
                   jax.ShapeDtypeStruct((n, kb), jnp.float32)],
    )(parts, xroot, wa, wb)


def _combine(parts, hroot, nb):
    n, d = hroot.shape

    def body(p_ref, hr_ref, o_ref):
        o_ref[...] = p_ref[0] + p_ref[1] + hr_ref[...]

    z = lambda: jnp.int32(0)
    return pl.pallas_call(
        body,
        grid=(n // nb,),
        in_specs=[pl.BlockSpec((NC, nb, d), lambda i: (z(), i, z())),
                  pl.BlockSpec((nb, d), lambda i: (i, z()))],
        out_specs=pl.BlockSpec((nb, d), lambda i: (i, z())),
        out_shape=jax.ShapeDtypeStruct((n, d), jnp.float32),
    )(parts, hroot)


def _jax_hist_norm(seg):
    cnt = jax.ops.segment_sum(jnp.ones((E,), jnp.float32), seg, num_segments=NR)
    return 1.0 / jnp.maximum(cnt[seg], 1.0)


def _jax_edge_agg(ridx, dst, norm, table, width):
    msg = table[ridx] * norm[:, None].astype(jnp.float32)
    agg = jax.ops.segment_sum(msg, dst, num_segments=NP)
    out = jnp.zeros((NC, NP, width), jnp.float32).at[0].set(agg)
    return out


def kernel(x, edge_index, edge_type, W1, Wroot1, W2, Wroot2):
    x = x.astype(jnp.float32)
    src = edge_index[0].astype(jnp.int32)
    dst = edge_index[1].astype(jnp.int32)
    et = edge_type.astype(jnp.int32)
    ridx = src * R + et
    seg = dst * R + et

    w1c = W1.astype(jnp.float32).transpose(1, 0, 2).reshape(D, R * H)
    w2c = W2.astype(jnp.float32).transpose(1, 0, 2).reshape(H, R * O)

    norm = _hist_norm(seg)
    hr1, xr = _mm2(x, w1c, Wroot1.astype(jnp.float32), 1000)
    agg1p = _edge_agg(ridx, dst, norm, hr1.reshape(NR, H), H)
    hr2, hroot = _relu_mm2(agg1p[:, :N], xr, w2c, Wroot2.astype(jnp.float32), 1000)
    gidx = lax.shift_right_logical(ridx, jnp.int32(3))
    sub = (ridx & jnp.int32(7)) * jnp.int32(O)
    agg2p = _edge_agg16(gidx, sub, dst, norm, hr2.reshape(NR // 8, 8 * O))
    return _combine(agg2p[:, :N, :O], hroot, 1000).astype(jnp.float64)
